# baseline (speedup 1.0000x reference)
import functools

import jax
import jax.numpy as jnp
from jax import lax
from jax.experimental import pallas as pl
from jax.experimental.pallas import tpu as pltpu
from jax.experimental.pallas import tpu_sc as plsc

_BLOCK_V = 4096
_VH_BLOCKS = 12
_VH = _VH_BLOCKS * _BLOCK_V


def _fuse_body(tablet_ref, w_ref, b_ref, g_ref, be_ref, out_ref):
    h = lax.dot_general(
        tablet_ref[...],
        w_ref[...],
        dimension_numbers=(((0,), (0,)), ((), ())),
        preferred_element_type=jnp.float32,
    )
    h = h + b_ref[...]
    mean = jnp.mean(h, axis=-1, keepdims=True)
    var = jnp.mean((h - mean) ** 2, axis=-1, keepdims=True)
    h = (h - mean) * lax.rsqrt(var + 1e-5)
    h = h * g_ref[...] + be_ref[...]
    out_ref[...] = jnp.maximum(h, 0.0)


def _fuse_half(tablet, W, b, gamma, beta, block_lo, v_rows):
    D, _ = tablet.shape
    E = W.shape[1]
    grid = (pl.cdiv(v_rows, _BLOCK_V),)
    return pl.pallas_call(
        _fuse_body,
        grid=grid,
        in_specs=[
            pl.BlockSpec((D, _BLOCK_V), lambda i, lo=block_lo: (0, i + lo)),
            pl.BlockSpec((D, E), lambda i: (0, 0)),
            pl.BlockSpec((1, E), lambda i: (0, 0)),
            pl.BlockSpec((1, E), lambda i: (0, 0)),
            pl.BlockSpec((1, E), lambda i: (0, 0)),
        ],
        out_specs=pl.BlockSpec((_BLOCK_V, E), lambda i: (i, 0)),
        out_shape=jax.ShapeDtypeStruct((v_rows, E), jnp.float32),
    )(tablet, W, b.reshape(1, E), gamma.reshape(1, E), beta.reshape(1, E))


def _make_phase_gather(E, per, cap, nbuf=5):
    info = plsc.get_sparse_core_info()
    nc, ns = info.num_cores, info.num_subcores
    mesh = plsc.VectorSubcoreMesh(core_axis_name="c", subcore_axis_name="s")
    lsz = per + 256

    @functools.partial(
        pl.kernel,
        mesh=mesh,
        out_type=(),
        scratch_types=[
            pltpu.VMEM((lsz,), jnp.int32),
            pltpu.VMEM((lsz // 128, 128), jnp.int32),
            pltpu.VMEM((128,), jnp.int32),
            pltpu.VMEM((nbuf, 128, E), jnp.float32),
        ]
        + [pltpu.SemaphoreType.DMA] * (2 * nbuf),
    )
    def gather(
        fused_hbm, sid_hbm, pos_hbm, nch_hbm, out_hbm, sid_v, pos_v, nch_v, rows, *sems
    ):
        gsems, ssems = sems[:nbuf], sems[nbuf:]
        wid = lax.axis_index("s") * nc + lax.axis_index("c")
        pltpu.sync_copy(sid_hbm.at[wid], sid_v)
        pltpu.sync_copy(pos_hbm.at[wid], pos_v)
        pltpu.sync_copy(nch_hbm.at[wid], nch_v)
        nch = nch_v[pl.ds(0, 16)][0]

        def gat(c, j):
            return pltpu.make_async_copy(
                fused_hbm.at[sid_v.at[pl.ds(c * 128, 128)]], rows.at[j], gsems[j]
            )

        def sca(c, j):
            return pltpu.make_async_copy(rows.at[j], out_hbm.at[pos_v.at[c]], ssems[j])

        for j in range(nbuf):

            @pl.when(nch > j)
            def _(j=j):
                gat(j, j).start()

        def mbody(g, carry):
            for j in range(nbuf):
                c = g * nbuf + j

                @pl.when(c < nch)
                def _(c=c, j=j):
                    gat(c, j).wait()
                    sca(c, j).start()

                    @pl.when(c + nbuf < nch)
                    def _(c=c, j=j):
                        sca(c, j).wait()
                        gat(c + nbuf, j).start()

            return carry

        lax.fori_loop(0, (cap + nbuf - 1) // nbuf, mbody, 0)

        for j in range(nbuf):

            @pl.when(nch > j)
            def _(j=j):
                sca(0, j).wait()

    return gather


def kernel(token_ids, table, W, b, gamma, beta):
    B, S = token_ids.shape
    V = table.shape[0]
    E = W.shape[1]
    N = B * S
    nw = 32
    per = N // nw
    cap = per // 128
    lsz = per + 256

    tablet = table.T
    fused_a = _fuse_half(tablet, W, b, gamma, beta, 0, _VH)
    fused_b = _fuse_half(tablet, W, b, gamma, beta, _VH_BLOCKS, V - _VH)

    ids = token_ids.T.reshape(nw, per).astype(jnp.int32)
    m = ids >= _VH
    sid = jnp.where(m, ids - _VH, ids)
    pos = jnp.arange(N, dtype=jnp.int32).reshape(nw, per)
    cum_a = jnp.cumsum((~m).astype(jnp.int32), axis=1)
    cum_b = jnp.cumsum(m.astype(jnp.int32), axis=1)
    n_a = cum_a[:, -1:]
    n_b = per - n_a
    dst_a = jnp.where(m, per + 128, cum_a - 1)
    dst_b = jnp.where(m, cum_b - 1, per + 128)
    j0_a = jnp.argmin(m, axis=1, keepdims=True)
    j0_b = jnp.argmax(m, axis=1, keepdims=True)

    def _defaults(src, j0):
        return jnp.broadcast_to(jnp.take_along_axis(src, j0, axis=1), (nw, lsz))

    sid_a = jnp.put_along_axis(_defaults(sid, j0_a), dst_a, sid, axis=1, inplace=False)
    pos_a = jnp.put_along_axis(_defaults(pos, j0_a), dst_a, pos, axis=1, inplace=False)
    sid_b = jnp.put_along_axis(_defaults(sid, j0_b), dst_b, sid, axis=1, inplace=False)
    pos_b = jnp.put_along_axis(_defaults(pos, j0_b), dst_b, pos, axis=1, inplace=False)
    nch_a = jnp.broadcast_to((n_a + 127) // 128, (nw, 128))
    nch_b = jnp.broadcast_to((n_b + 127) // 128, (nw, 128))

    out_ref = jax.new_ref(lax.empty((N, E), jnp.float32))
    g = _make_phase_gather(E, per, cap)
    g(fused_a, sid_a, pos_a.reshape(nw, lsz // 128, 128), nch_a, out_ref)
    g(fused_b, sid_b, pos_b.reshape(nw, lsz // 128, 128), nch_b, out_ref)
    return out_ref[...].reshape(S, B, E).transpose(1, 0, 2)


# speedup vs baseline: 5.7716x; 5.7716x over previous
import functools

import jax
import jax.numpy as jnp
from jax import lax
from jax.experimental import pallas as pl
from jax.experimental.pallas import tpu as pltpu
from jax.experimental.pallas import tpu_sc as plsc

_BLOCK_V = 4096
_VH_BLOCKS = 12
_VH = _VH_BLOCKS * _BLOCK_V


def _fuse_body(tablet_ref, w_ref, b_ref, g_ref, be_ref, out_ref):
    h = lax.dot_general(
        tablet_ref[...],
        w_ref[...],
        dimension_numbers=(((0,), (0,)), ((), ())),
        preferred_element_type=jnp.float32,
    )
    h = h + b_ref[...]
    mean = jnp.mean(h, axis=-1, keepdims=True)
    var = jnp.mean((h - mean) ** 2, axis=-1, keepdims=True)
    h = (h - mean) * lax.rsqrt(var + 1e-5)
    h = h * g_ref[...] + be_ref[...]
    out_ref[...] = jnp.maximum(h, 0.0)


def _fuse_half(tablet, W, b, gamma, beta, block_lo, v_rows):
    D, _ = tablet.shape
    E = W.shape[1]
    grid = (pl.cdiv(v_rows, _BLOCK_V),)
    return pl.pallas_call(
        _fuse_body,
        grid=grid,
        in_specs=[
            pl.BlockSpec((D, _BLOCK_V), lambda i, lo=block_lo: (0, i + lo)),
            pl.BlockSpec((D, E), lambda i: (0, 0)),
            pl.BlockSpec((1, E), lambda i: (0, 0)),
            pl.BlockSpec((1, E), lambda i: (0, 0)),
            pl.BlockSpec((1, E), lambda i: (0, 0)),
        ],
        out_specs=pl.BlockSpec((_BLOCK_V, E), lambda i: (i, 0)),
        out_shape=jax.ShapeDtypeStruct((v_rows, E), jnp.float32),
    )(tablet, W, b.reshape(1, E), gamma.reshape(1, E), beta.reshape(1, E))


def _make_phase_gather(E, per, cap, nbuf=5):
    info = plsc.get_sparse_core_info()
    nc, ns = info.num_cores, info.num_subcores
    mesh = plsc.VectorSubcoreMesh(core_axis_name="c", subcore_axis_name="s")
    lsz = per + 256

    @functools.partial(
        pl.kernel,
        mesh=mesh,
        out_type=(),
        scratch_types=[
            pltpu.VMEM((lsz,), jnp.int32),
            pltpu.VMEM((lsz // 128, 128), jnp.int32),
            pltpu.VMEM((128,), jnp.int32),
            pltpu.VMEM((nbuf, 128, E), jnp.float32),
        ]
        + [pltpu.SemaphoreType.DMA] * (2 * nbuf),
    )
    def gather(
        fused_hbm, sid_hbm, pos_hbm, nch_hbm, out_hbm, sid_v, pos_v, nch_v, rows, *sems
    ):
        gsems, ssems = sems[:nbuf], sems[nbuf:]
        wid = lax.axis_index("s") * nc + lax.axis_index("c")
        pltpu.sync_copy(sid_hbm.at[wid], sid_v)
        pltpu.sync_copy(pos_hbm.at[wid], pos_v)
        pltpu.sync_copy(nch_hbm.at[wid], nch_v)
        nch = nch_v[pl.ds(0, 16)][0]

        def gat(c, j):
            return pltpu.make_async_copy(
                fused_hbm.at[sid_v.at[pl.ds(c * 128, 128)]], rows.at[j], gsems[j]
            )

        def sca(c, j):
            return pltpu.make_async_copy(rows.at[j], out_hbm.at[pos_v.at[c]], ssems[j])

        for j in range(nbuf):

            @pl.when(nch > j)
            def _(j=j):
                gat(j, j).start()

        def mbody(g, carry):
            for j in range(nbuf):
                c = g * nbuf + j

                @pl.when(c < nch)
                def _(c=c, j=j):
                    gat(c, j).wait()
                    sca(c, j).start()

                    @pl.when(c + nbuf < nch)
                    def _(c=c, j=j):
                        sca(c, j).wait()
                        gat(c + nbuf, j).start()

            return carry

        lax.fori_loop(0, (cap + nbuf - 1) // nbuf, mbody, 0)

        for j in range(nbuf):

            @pl.when(nch > j)
            def _(j=j):
                sca(0, j).wait()

    return gather


def kernel(token_ids, table, W, b, gamma, beta):
    B, S = token_ids.shape
    V = table.shape[0]
    E = W.shape[1]
    N = B * S
    nw = 32
    per = N // nw
    cap = per // 128
    lsz = per + 256

    tablet = table.T
    fused_a = _fuse_half(tablet, W, b, gamma, beta, 0, _VH)
    fused_b = _fuse_half(tablet, W, b, gamma, beta, _VH_BLOCKS, V - _VH)

    ids = token_ids.T.reshape(nw, per).astype(jnp.int32)
    m = ids >= _VH
    sid = jnp.where(m, ids - _VH, ids)
    pos = jnp.arange(N, dtype=jnp.int32).reshape(nw, per)
    n_a = jnp.sum((~m).astype(jnp.int32), axis=1, keepdims=True)
    n_b = per - n_a
    perm_a = jnp.argsort(m, axis=1, stable=True).astype(jnp.int32)
    perm_b = jnp.argsort(~m, axis=1, stable=True).astype(jnp.int32)

    def _lists(perm, own_mask):
        own = jnp.take_along_axis(own_mask, perm, axis=1)
        sid_s = jnp.take_along_axis(sid, perm, axis=1)
        sid_s = jnp.where(own, sid_s, sid_s[:, 0:1])
        pos_s = jnp.take_along_axis(pos, perm, axis=1)
        pos_s = jnp.where(own, pos_s, pos_s[:, 0:1])
        pad = jnp.zeros((nw, lsz - per), jnp.int32)
        sid_l = jnp.concatenate([sid_s, pad + sid_s[:, 0:1]], axis=1)
        pos_l = jnp.concatenate([pos_s, pad + pos_s[:, 0:1]], axis=1)
        return sid_l, pos_l

    sid_a, pos_a = _lists(perm_a, ~m)
    sid_b, pos_b = _lists(perm_b, m)
    nch_a = jnp.broadcast_to((n_a + 127) // 128, (nw, 128))
    nch_b = jnp.broadcast_to((n_b + 127) // 128, (nw, 128))

    out_ref = jax.new_ref(lax.empty((N, E), jnp.float32))
    g = _make_phase_gather(E, per, cap)
    g(fused_a, sid_a, pos_a.reshape(nw, lsz // 128, 128), nch_a, out_ref)
    g(fused_b, sid_b, pos_b.reshape(nw, lsz // 128, 128), nch_b, out_ref)
    return out_ref[...].reshape(S, B, E).transpose(1, 0, 2)


# speedup vs baseline: 19.3640x; 3.3550x over previous
import functools

import jax
import jax.numpy as jnp
from jax import lax
from jax.experimental import pallas as pl
from jax.experimental.pallas import tpu as pltpu
from jax.experimental.pallas import tpu_sc as plsc


def _fuse_body(tablet_ref, w_ref, b_ref, g_ref, be_ref, out_ref):
    h = lax.dot_general(
        tablet_ref[...],
        w_ref[...],
        dimension_numbers=(((0,), (0,)), ((), ())),
        preferred_element_type=jnp.float32,
    )
    h = h + b_ref[...]
    mean = jnp.mean(h, axis=-1, keepdims=True)
    var = jnp.mean((h - mean) ** 2, axis=-1, keepdims=True)
    h = (h - mean) * lax.rsqrt(var + 1e-5)
    h = h * g_ref[...] + be_ref[...]
    out_ref[...] = jnp.maximum(h, 0.0)


def _fuse_table(table, W, b, gamma, beta, block_v=8192):
    V, D = table.shape
    E = W.shape[1]
    tablet = table.T
    grid = (pl.cdiv(V, block_v),)
    return pl.pallas_call(
        _fuse_body,
        grid=grid,
        in_specs=[
            pl.BlockSpec((D, block_v), lambda i: (0, i)),
            pl.BlockSpec((D, E), lambda i: (0, 0)),
            pl.BlockSpec((1, E), lambda i: (0, 0)),
            pl.BlockSpec((1, E), lambda i: (0, 0)),
            pl.BlockSpec((1, E), lambda i: (0, 0)),
        ],
        out_specs=pl.BlockSpec((block_v, E), lambda i: (i, 0)),
        out_shape=jax.ShapeDtypeStruct((V, E), jnp.float32),
    )(tablet, W, b.reshape(1, E), gamma.reshape(1, E), beta.reshape(1, E))


def _make_gather(E, n_total, chunk=128, nbuf=5):
    info = plsc.get_sparse_core_info()
    nc, ns = info.num_cores, info.num_subcores
    nw = nc * ns
    n_chunks = n_total // (nw * chunk)
    assert n_chunks * nw * chunk == n_total and n_chunks % nbuf == 0
    n_groups = n_chunks // nbuf
    mesh = plsc.VectorSubcoreMesh(core_axis_name="c", subcore_axis_name="s")

    @functools.partial(
        pl.kernel,
        mesh=mesh,
        out_type=jax.ShapeDtypeStruct((n_total, E), jnp.float32),
        scratch_types=[
            pltpu.VMEM((n_chunks, chunk), jnp.int32),
            pltpu.VMEM((nbuf, chunk, E), jnp.float32),
        ]
        + [pltpu.SemaphoreType.DMA] * (2 * nbuf),
    )
    def gather(table_hbm, idx_hbm, out_hbm, idx_v, rows_v, *sems):
        gsems, ssems = sems[:nbuf], sems[nbuf:]
        wid = lax.axis_index("s") * nc + lax.axis_index("c")
        pltpu.sync_copy(idx_hbm.at[wid], idx_v)

        def out_at(c):
            return out_hbm.at[pl.ds((wid * n_chunks + c) * chunk, chunk)]

        for j in range(nbuf):
            pltpu.async_copy(table_hbm.at[idx_v.at[j]], rows_v.at[j], gsems[j])

        def body(g, carry):
            for j in range(nbuf):
                c = g * nbuf + j
                pltpu.make_async_copy(
                    table_hbm.at[idx_v.at[c]], rows_v.at[j], gsems[j]
                ).wait()
                pltpu.async_copy(rows_v.at[j], out_at(c), ssems[j])
            for j in range(nbuf):
                c_next = (g + 1) * nbuf + j

                @pl.when(c_next < n_chunks)
                def _():
                    pltpu.make_async_copy(
                        rows_v.at[j], out_at(c_next - nbuf), ssems[j]
                    ).wait()
                    pltpu.async_copy(
                        table_hbm.at[idx_v.at[c_next]], rows_v.at[j], gsems[j]
                    )

            return carry

        lax.fori_loop(0, n_groups, body, 0)
        for j in range(nbuf):
            c = (n_groups - 1) * nbuf + j
            pltpu.make_async_copy(rows_v.at[j], out_at(c), ssems[j]).wait()

    def run(fused, idx_flat):
        idx3 = idx_flat.reshape(nw, n_chunks, chunk)
        return gather(fused, idx3)

    return run


def kernel(token_ids, table, W, b, gamma, beta):
    B, S = token_ids.shape
    E = W.shape[1]
    fused = _fuse_table(table, W, b, gamma, beta)
    idx_flat = token_ids.T.reshape(-1).astype(jnp.int32)
    out = _make_gather(E, B * S)(fused, idx_flat)
    return out.reshape(S, B, E).transpose(1, 0, 2)
